# Initial kernel scaffold; baseline (speedup 1.0000x reference)
import jax
import jax.numpy as jnp
from jax.experimental import pallas as pl


def kernel(node_feats, edge_feats, edge_index, edge_indices, W1, b1, W2, b2, conv_bias, Wc1, bc1, Wc2, bc2):
    raise NotImplementedError("write your pallas kernel here")


# speedup vs baseline: 2.1691x; 2.1691x over previous
import functools

import jax
import jax.numpy as jnp
from jax import lax
from jax.experimental import pallas as pl
from jax.experimental.pallas import tpu as pltpu
from jax.experimental.pallas import tpu_sc as plsc

NC, NS = 2, 16
NW = NC * NS
CH = 128


def _mesh():
    return plsc.VectorSubcoreMesh(core_axis_name="c", subcore_axis_name="s")


_SC_PARAMS = pltpu.CompilerParams(use_tc_tiling_on_sc=False)


def _wid():
    return lax.axis_index("s") * NC + lax.axis_index("c")


def _gather_rows_kernel(table_hbm, idx_hbm, out_hbm, idx_v, rows_v, sem):
    wid = _wid()
    k = idx_v.shape[0]
    pltpu.sync_copy(idx_hbm.at[wid], idx_v)
    for j in range(k):
        pltpu.async_copy(table_hbm.at[idx_v.at[j]], rows_v, sem).wait()
        pltpu.sync_copy(rows_v, out_hbm.at[pl.ds((wid * k + j) * CH, CH)])


def _make_gather_rows(n_rows, width, k, dtype):
    return functools.partial(
        pl.kernel,
        out_type=jax.ShapeDtypeStruct((NW * k * CH, width), dtype),
        mesh=_mesh(),
        scratch_types=[
            pltpu.VMEM((k, CH), jnp.int32),
            pltpu.VMEM((CH, width), dtype),
            pltpu.SemaphoreType.DMA,
        ],
        compiler_params=_SC_PARAMS,
    )(_gather_rows_kernel)


def _scatter_add_kernel(aug_hbm, dst_hbm, zeros_hbm, out_hbm, idx_v, rows_v,
                        acc_sh):
    c = lax.axis_index("c")
    s = lax.axis_index("s")
    wid = s * NC + c
    k = idx_v.shape[0]
    n = acc_sh.shape[0]
    rpt = n // NS
    pltpu.sync_copy(zeros_hbm.at[pl.ds(s * rpt, rpt)],
                    acc_sh.at[pl.ds(s * rpt, rpt)])
    pltpu.sync_copy(dst_hbm.at[wid], idx_v)
    plsc.subcore_barrier()
    for j in range(k):
        pltpu.sync_copy(aug_hbm.at[pl.ds((wid * k + j) * CH, CH)], rows_v)
        pltpu.sync_copy(rows_v, acc_sh.at[idx_v.at[j]], add=True)
    plsc.subcore_barrier()
    pltpu.sync_copy(acc_sh.at[pl.ds(s * rpt, rpt)],
                    out_hbm.at[c].at[pl.ds(s * rpt, rpt)])


def _idx_gather_kernel(src_hbm, dst_hbm, ei_hbm, sidx_hbm, didx_hbm,
                       table_v, ei_v, out_v):
    wid = _wid()
    pt = ei_v.shape[0]
    pltpu.sync_copy(ei_hbm.at[pl.ds(wid * pt, pt)], ei_v)
    for table, out in ((src_hbm, sidx_hbm), (dst_hbm, didx_hbm)):
        pltpu.sync_copy(table, table_v)
        for j in range(pt // 16):
            idx16 = ei_v[pl.ds(j * 16, 16)]
            out_v[pl.ds(j * 16, 16)] = plsc.load_gather(table_v, [idx16])
        pltpu.sync_copy(out_v, out.at[pl.ds(wid * pt, pt)])


def _mlp_msg_kernel(ef_ref, xs_ref, v_ref, w1_ref, b1_ref, w2_ref, b2_ref,
                    r_ref, st_ref, u_ref, out_ref):
    t = jnp.maximum(
        jnp.dot(ef_ref[...], w1_ref[...], preferred_element_type=jnp.float32)
        + b1_ref[...], 0.0)
    we = jnp.dot(t, w2_ref[...],
                 preferred_element_type=jnp.float32) + b2_ref[...]
    xe = jnp.dot(xs_ref[...], r_ref[...], preferred_element_type=jnp.float32)
    msg = jnp.dot(we * xe, st_ref[...], preferred_element_type=jnp.float32)
    v = v_ref[...]
    out_ref[...] = msg * v + v * u_ref[...]


def _mean_relu_kernel(bias_ref, p0_ref, p1_ref, h_ref):
    srow = p0_ref[0] + p1_ref[0]
    agg = srow[:, :16]
    deg = srow[:, 16:17]
    h_ref[...] = jnp.maximum(agg / jnp.maximum(deg, 1.0) + bias_ref[...], 0.0)


def _classifier_kernel(sh_ref, dh_ref, ef_ref, w1s_ref, w1d_ref, w1e_ref,
                       bc1_ref, wc2_ref, bc2_ref, out_ref):
    z = (jnp.dot(sh_ref[...], w1s_ref[...], preferred_element_type=jnp.float32)
         + jnp.dot(dh_ref[...], w1d_ref[...],
                   preferred_element_type=jnp.float32)
         + jnp.dot(ef_ref[...], w1e_ref[...],
                   preferred_element_type=jnp.float32)
         + bc1_ref[...])
    out_ref[...] = jnp.dot(jnp.maximum(z, 0.0), wc2_ref[...],
                           preferred_element_type=jnp.float32) + bc2_ref[...]


def kernel(node_feats, edge_feats, edge_index, edge_indices, W1, b1, W2, b2,
           conv_bias, Wc1, bc1, Wc2, bc2):
    n, in_f = node_feats.shape
    e, ef_f = edge_feats.shape
    h_f = conv_bias.shape[0]
    out_f = Wc2.shape[1]
    nsup = edge_indices.shape[0]

    k_e = -(-e // (NW * CH))
    e_pad = NW * k_e * CH
    k_s = -(-nsup // (NW * CH))
    nsup_pad = NW * k_s * CH
    pt = k_s * CH

    src = edge_index[0]
    dst = edge_index[1]
    src_p = jnp.pad(src, (0, e_pad - e))
    dst_p = jnp.pad(dst, (0, e_pad - e))
    ef_p = jnp.pad(edge_feats, ((0, e_pad - e), (0, 0)))
    ei_p = jnp.pad(edge_indices, (0, nsup_pad - nsup))
    valid = (jnp.arange(e_pad) < e).astype(jnp.float32)[:, None]

    cols = jnp.arange(in_f * h_f)
    r_m = (jnp.arange(in_f)[:, None] == cols[None, :] // h_f
           ).astype(jnp.float32)
    st_m = (cols[:, None] % h_f == jnp.arange(32)[None, :]
            ).astype(jnp.float32)
    u_row = (jnp.arange(32) == h_f).astype(jnp.float32)[None, :]

    xs = _make_gather_rows(n, in_f, k_e, jnp.float32)(
        node_feats, src_p.reshape(NW, k_e, CH))

    be = 1024
    aug = pl.pallas_call(
        _mlp_msg_kernel,
        grid=(e_pad // be,),
        in_specs=[
            pl.BlockSpec((be, ef_f), lambda i: (i, 0)),
            pl.BlockSpec((be, in_f), lambda i: (i, 0)),
            pl.BlockSpec((be, 1), lambda i: (i, 0)),
            pl.BlockSpec(W1.shape, lambda i: (0, 0)),
            pl.BlockSpec((1, b1.shape[0]), lambda i: (0, 0)),
            pl.BlockSpec(W2.shape, lambda i: (0, 0)),
            pl.BlockSpec((1, b2.shape[0]), lambda i: (0, 0)),
            pl.BlockSpec(r_m.shape, lambda i: (0, 0)),
            pl.BlockSpec(st_m.shape, lambda i: (0, 0)),
            pl.BlockSpec(u_row.shape, lambda i: (0, 0)),
        ],
        out_specs=pl.BlockSpec((be, 32), lambda i: (i, 0)),
        out_shape=jax.ShapeDtypeStruct((e_pad, 32), jnp.float32),
    )(ef_p, xs, valid, W1, b1[None, :], W2, b2[None, :], r_m, st_m, u_row)

    partials = pl.kernel(
        _scatter_add_kernel,
        out_type=jax.ShapeDtypeStruct((NC, n, 32), jnp.float32),
        mesh=_mesh(),
        scratch_types=[
            pltpu.VMEM((k_e, CH), jnp.int32),
            pltpu.VMEM((CH, 32), jnp.float32),
            pltpu.VMEM_SHARED((n, 32), jnp.float32),
        ],
        compiler_params=_SC_PARAMS,
    )(aug, dst_p.reshape(NW, k_e, CH), jnp.zeros((n, 32), jnp.float32))

    bn = 2000
    h = pl.pallas_call(
        _mean_relu_kernel,
        grid=(n // bn,),
        in_specs=[
            pl.BlockSpec((1, h_f), lambda i: (0, 0)),
            pl.BlockSpec((1, bn, 32), lambda i: (0, i, 0)),
            pl.BlockSpec((1, bn, 32), lambda i: (1, i, 0)),
        ],
        out_specs=pl.BlockSpec((bn, h_f), lambda i: (i, 0)),
        out_shape=jax.ShapeDtypeStruct((n, h_f), jnp.float32),
    )(conv_bias[None, :], partials, partials)

    s_idx, d_idx = pl.kernel(
        _idx_gather_kernel,
        out_type=(jax.ShapeDtypeStruct((nsup_pad,), jnp.int32),
                  jax.ShapeDtypeStruct((nsup_pad,), jnp.int32)),
        mesh=_mesh(),
        scratch_types=[
            pltpu.VMEM((e_pad,), jnp.int32),
            pltpu.VMEM((pt,), jnp.int32),
            pltpu.VMEM((pt,), jnp.int32),
        ],
        compiler_params=pltpu.CompilerParams(needs_layout_passes=False),
    )(src_p, dst_p, ei_p)

    sh = _make_gather_rows(n, h_f, k_s, jnp.float32)(
        h, s_idx.reshape(NW, k_s, CH))
    dh = _make_gather_rows(n, h_f, k_s, jnp.float32)(
        h, d_idx.reshape(NW, k_s, CH))
    efc = _make_gather_rows(e, ef_f, k_s, jnp.float32)(
        edge_feats, ei_p.reshape(NW, k_s, CH))

    bs = 1024
    logits = pl.pallas_call(
        _classifier_kernel,
        grid=(nsup_pad // bs,),
        in_specs=[
            pl.BlockSpec((bs, h_f), lambda i: (i, 0)),
            pl.BlockSpec((bs, h_f), lambda i: (i, 0)),
            pl.BlockSpec((bs, ef_f), lambda i: (i, 0)),
            pl.BlockSpec((h_f, h_f), lambda i: (0, 0)),
            pl.BlockSpec((h_f, h_f), lambda i: (0, 0)),
            pl.BlockSpec((ef_f, h_f), lambda i: (0, 0)),
            pl.BlockSpec((1, h_f), lambda i: (0, 0)),
            pl.BlockSpec(Wc2.shape, lambda i: (0, 0)),
            pl.BlockSpec((1, out_f), lambda i: (0, 0)),
        ],
        out_specs=pl.BlockSpec((bs, out_f), lambda i: (i, 0)),
        out_shape=jax.ShapeDtypeStruct((nsup_pad, out_f), jnp.float32),
    )(sh, dh, efc, Wc1[:h_f], Wc1[h_f:2 * h_f], Wc1[2 * h_f:],
      bc1[None, :], Wc2, bc2[None, :])

    return logits[:nsup]


# speedup vs baseline: 2.3923x; 1.1029x over previous
import jax
import jax.numpy as jnp
from jax import lax
from jax.experimental import pallas as pl
from jax.experimental.pallas import tpu as pltpu
from jax.experimental.pallas import tpu_sc as plsc

NC, NS = 2, 16
NW = NC * NS
CH = 128


def _mesh():
    return plsc.VectorSubcoreMesh(core_axis_name="c", subcore_axis_name="s")


_SC_PARAMS = pltpu.CompilerParams(use_tc_tiling_on_sc=False)


def _wid():
    return lax.axis_index("s") * NC + lax.axis_index("c")


def _gather_rows_kernel(table_hbm, idx_hbm, out_hbm, idx_v, rows_v, sem):
    wid = _wid()
    k = idx_v.shape[0]
    pltpu.sync_copy(idx_hbm.at[wid], idx_v)
    for j in range(k):
        pltpu.async_copy(table_hbm.at[idx_v.at[j]], rows_v, sem).wait()
        pltpu.sync_copy(rows_v, out_hbm.at[pl.ds((wid * k + j) * CH, CH)])


def _scatter_add_kernel(aug_hbm, dst_hbm, zeros_hbm, ones_hbm, out_hbm,
                        idx_v, rows_v, ones_v, acc_msg, acc_deg):
    c = lax.axis_index("c")
    s = lax.axis_index("s")
    wid = s * NC + c
    k = idx_v.shape[0]
    n = out_hbm.shape[1]
    rpt = n // NS
    pltpu.sync_copy(zeros_hbm.at[pl.ds(0, rpt)],
                    acc_msg.at[pl.ds(s * rpt, rpt)])
    pltpu.sync_copy(zeros_hbm.at[pl.ds(0, rpt)],
                    acc_deg.at[pl.ds(s * rpt, rpt)])
    @pl.when(s == 0)
    def _():
        pad = acc_msg.shape[0] - n
        pltpu.sync_copy(zeros_hbm.at[pl.ds(0, pad)],
                        acc_msg.at[pl.ds(n, pad)])
        pltpu.sync_copy(zeros_hbm.at[pl.ds(0, pad)],
                        acc_deg.at[pl.ds(n, pad)])
    pltpu.sync_copy(dst_hbm.at[wid], idx_v)
    pltpu.sync_copy(ones_hbm, ones_v)
    plsc.subcore_barrier()
    for j in range(k):
        pltpu.sync_copy(
            aug_hbm.at[pl.ds((wid * k + j) * CH, CH), pl.ds(0, 16)], rows_v)
        pltpu.sync_copy(rows_v, acc_msg.at[idx_v.at[j]], add=True)
        pltpu.sync_copy(ones_v, acc_deg.at[idx_v.at[j]], add=True)
    plsc.subcore_barrier()
    pltpu.sync_copy(acc_msg.at[pl.ds(s * rpt, rpt)],
                    out_hbm.at[c, pl.ds(s * rpt, rpt), pl.ds(0, 16)])
    pltpu.sync_copy(acc_deg.at[pl.ds(s * rpt, rpt)],
                    out_hbm.at[c, pl.ds(s * rpt, rpt), pl.ds(16, 16)])


def _idx_gather_kernel(src_hbm, dst_hbm, ei_hbm, sidx_hbm, didx_hbm,
                       table_v, ei_v, out_v):
    wid = _wid()
    pt = ei_v.shape[0]
    pltpu.sync_copy(ei_hbm.at[pl.ds(wid * pt, pt)], ei_v)
    for table, out in ((src_hbm, sidx_hbm), (dst_hbm, didx_hbm)):
        pltpu.sync_copy(table, table_v)
        for j in range(pt // 16):
            idx16 = ei_v[pl.ds(j * 16, 16)]
            out_v[pl.ds(j * 16, 16)] = plsc.load_gather(table_v, [idx16])
        pltpu.sync_copy(out_v, out.at[pl.ds(wid * pt, pt)])


def _cls_gather_kernel(h_hbm, aug_hbm, si_hbm, di_hbm, ei_hbm, out_hbm,
                       si_v, di_v, ei_v, rows16_v, rows32_v, sem):
    wid = _wid()
    k = si_v.shape[0]
    pltpu.sync_copy(si_hbm.at[wid], si_v)
    pltpu.sync_copy(di_hbm.at[wid], di_v)
    pltpu.sync_copy(ei_hbm.at[wid], ei_v)
    for j in range(k):
        rows = pl.ds((wid * k + j) * CH, CH)
        pltpu.async_copy(h_hbm.at[si_v.at[j]], rows16_v, sem).wait()
        pltpu.sync_copy(rows16_v, out_hbm.at[rows, pl.ds(0, 16)])
        pltpu.async_copy(h_hbm.at[di_v.at[j]], rows16_v, sem).wait()
        pltpu.sync_copy(rows16_v, out_hbm.at[rows, pl.ds(16, 16)])
        pltpu.async_copy(aug_hbm.at[ei_v.at[j]], rows32_v, sem).wait()
        pltpu.sync_copy(rows32_v.at[pl.ds(0, CH), pl.ds(16, 16)],
                        out_hbm.at[rows, pl.ds(32, 16)])


def _make_mlp_msg_kernel(be, e_valid):
    def body(ef_ref, xs_ref, w1_ref, b1_ref, w2_ref, b2_ref, r_ref, st_ref,
             wc1e_ref, out_ref):
        pid = pl.program_id(0)
        gid = jax.lax.broadcasted_iota(jnp.int32, (be, 1), 0) + pid * be
        v = (gid < e_valid).astype(jnp.float32)
        ef = ef_ref[...] * v
        t = jnp.maximum(
            jnp.dot(ef, w1_ref[...], preferred_element_type=jnp.float32)
            + b1_ref[...], 0.0)
        we = jnp.dot(t.astype(jnp.bfloat16), w2_ref[...],
                     preferred_element_type=jnp.float32) + b2_ref[...]
        xe = jnp.dot(xs_ref[...], r_ref[...],
                     preferred_element_type=jnp.float32)
        msg = jnp.dot(we * xe, st_ref[...],
                      preferred_element_type=jnp.float32) * v
        ec = jnp.dot(ef, wc1e_ref[...], preferred_element_type=jnp.float32)
        out_ref[...] = jnp.concatenate([msg, ec], axis=1)
    return body


def _mean_relu_kernel(bias_ref, p0_ref, p1_ref, h_ref):
    srow = p0_ref[0] + p1_ref[0]
    agg = srow[:, :16]
    deg = srow[:, 16:17]
    h_ref[...] = jnp.maximum(agg / jnp.maximum(deg, 1.0) + bias_ref[...], 0.0)


def _classifier_kernel(cls_ref, w1s_ref, w1d_ref, bc1_ref, wc2_ref, bc2_ref,
                       out_ref):
    cls = cls_ref[...]
    z = (jnp.dot(cls[:, 0:16], w1s_ref[...],
                 preferred_element_type=jnp.float32)
         + jnp.dot(cls[:, 16:32], w1d_ref[...],
                   preferred_element_type=jnp.float32)
         + cls[:, 32:48] + bc1_ref[...])
    out_ref[...] = jnp.dot(jnp.maximum(z, 0.0), wc2_ref[...],
                           preferred_element_type=jnp.float32) + bc2_ref[...]


def kernel(node_feats, edge_feats, edge_index, edge_indices, W1, b1, W2, b2,
           conv_bias, Wc1, bc1, Wc2, bc2):
    n, in_f = node_feats.shape
    e, ef_f = edge_feats.shape
    h_f = conv_bias.shape[0]
    out_f = Wc2.shape[1]
    nsup = edge_indices.shape[0]

    k_e = -(-e // (NW * CH))
    e_pad = NW * k_e * CH
    k_s = -(-nsup // (NW * CH))
    nsup_pad = NW * k_s * CH
    pt = k_s * CH
    n_acc = n + 16

    src = edge_index[0]
    dst = edge_index[1]
    src_p = jnp.pad(src, (0, e_pad - e))
    dst_p = jnp.pad(dst, (0, e_pad - e), constant_values=n)
    ei_p = jnp.pad(edge_indices, (0, nsup_pad - nsup))

    cols = jnp.arange(in_f * h_f)
    r_m = (jnp.arange(in_f)[:, None] == cols[None, :] // h_f
           ).astype(jnp.float32)
    st_m = (cols[:, None] % h_f == jnp.arange(h_f)[None, :]
            ).astype(jnp.float32)

    xs = pl.kernel(
        _gather_rows_kernel,
        out_type=jax.ShapeDtypeStruct((e_pad, in_f), jnp.float32),
        mesh=_mesh(),
        scratch_types=[
            pltpu.VMEM((k_e, CH), jnp.int32),
            pltpu.VMEM((CH, in_f), jnp.float32),
            pltpu.SemaphoreType.DMA,
        ],
        compiler_params=_SC_PARAMS,
    )(node_feats, src_p.reshape(NW, k_e, CH))

    be = 3072
    aug = pl.pallas_call(
        _make_mlp_msg_kernel(be, e),
        grid=(e_pad // be,),
        in_specs=[
            pl.BlockSpec((be, ef_f), lambda i: (i, 0)),
            pl.BlockSpec((be, in_f), lambda i: (i, 0)),
            pl.BlockSpec(W1.shape, lambda i: (0, 0)),
            pl.BlockSpec((1, b1.shape[0]), lambda i: (0, 0)),
            pl.BlockSpec(W2.shape, lambda i: (0, 0)),
            pl.BlockSpec((1, b2.shape[0]), lambda i: (0, 0)),
            pl.BlockSpec(r_m.shape, lambda i: (0, 0)),
            pl.BlockSpec(st_m.shape, lambda i: (0, 0)),
            pl.BlockSpec((ef_f, h_f), lambda i: (0, 0)),
        ],
        out_specs=pl.BlockSpec((be, 32), lambda i: (i, 0)),
        out_shape=jax.ShapeDtypeStruct((e_pad, 32), jnp.float32),
    )(edge_feats, xs, W1, b1[None, :], W2.astype(jnp.bfloat16),
      b2[None, :], r_m, st_m, Wc1[2 * h_f:])

    rpt = n // NS
    partials = pl.kernel(
        _scatter_add_kernel,
        out_type=jax.ShapeDtypeStruct((NC, n, 32), jnp.float32),
        mesh=_mesh(),
        scratch_types=[
            pltpu.VMEM((k_e, CH), jnp.int32),
            pltpu.VMEM((CH, 16), jnp.float32),
            pltpu.VMEM((CH, 16), jnp.float32),
            pltpu.VMEM_SHARED((n_acc, 16), jnp.float32),
            pltpu.VMEM_SHARED((n_acc, 16), jnp.float32),
        ],
        compiler_params=_SC_PARAMS,
    )(aug, dst_p.reshape(NW, k_e, CH), jnp.zeros((rpt, 16), jnp.float32),
      jnp.ones((CH, 16), jnp.float32))

    bn = 2000
    h = pl.pallas_call(
        _mean_relu_kernel,
        grid=(n // bn,),
        in_specs=[
            pl.BlockSpec((1, h_f), lambda i: (0, 0)),
            pl.BlockSpec((1, bn, 32), lambda i: (0, i, 0)),
            pl.BlockSpec((1, bn, 32), lambda i: (1, i, 0)),
        ],
        out_specs=pl.BlockSpec((bn, h_f), lambda i: (i, 0)),
        out_shape=jax.ShapeDtypeStruct((n, h_f), jnp.float32),
    )(conv_bias[None, :], partials, partials)

    s_idx, d_idx = pl.kernel(
        _idx_gather_kernel,
        out_type=(jax.ShapeDtypeStruct((nsup_pad,), jnp.int32),
                  jax.ShapeDtypeStruct((nsup_pad,), jnp.int32)),
        mesh=_mesh(),
        scratch_types=[
            pltpu.VMEM((e_pad,), jnp.int32),
            pltpu.VMEM((pt,), jnp.int32),
            pltpu.VMEM((pt,), jnp.int32),
        ],
        compiler_params=pltpu.CompilerParams(needs_layout_passes=False),
    )(src_p, dst_p, ei_p)

    cls_in = pl.kernel(
        _cls_gather_kernel,
        out_type=jax.ShapeDtypeStruct((nsup_pad, 48), jnp.float32),
        mesh=_mesh(),
        scratch_types=[
            pltpu.VMEM((k_s, CH), jnp.int32),
            pltpu.VMEM((k_s, CH), jnp.int32),
            pltpu.VMEM((k_s, CH), jnp.int32),
            pltpu.VMEM((CH, 16), jnp.float32),
            pltpu.VMEM((CH, 32), jnp.float32),
            pltpu.SemaphoreType.DMA,
        ],
        compiler_params=_SC_PARAMS,
    )(h, aug, s_idx.reshape(NW, k_s, CH), d_idx.reshape(NW, k_s, CH),
      ei_p.reshape(NW, k_s, CH))

    bs = 1024
    logits = pl.pallas_call(
        _classifier_kernel,
        grid=(nsup_pad // bs,),
        in_specs=[
            pl.BlockSpec((bs, 48), lambda i: (i, 0)),
            pl.BlockSpec((h_f, h_f), lambda i: (0, 0)),
            pl.BlockSpec((h_f, h_f), lambda i: (0, 0)),
            pl.BlockSpec((1, h_f), lambda i: (0, 0)),
            pl.BlockSpec(Wc2.shape, lambda i: (0, 0)),
            pl.BlockSpec((1, out_f), lambda i: (0, 0)),
        ],
        out_specs=pl.BlockSpec((bs, out_f), lambda i: (i, 0)),
        out_shape=jax.ShapeDtypeStruct((nsup_pad, out_f), jnp.float32),
    )(cls_in, Wc1[:h_f], Wc1[h_f:2 * h_f], bc1[None, :], Wc2, bc2[None, :])

    return logits[:nsup]


# speedup vs baseline: 2.5004x; 1.0452x over previous
import jax
import jax.numpy as jnp
from jax import lax
from jax.experimental import pallas as pl
from jax.experimental.pallas import tpu as pltpu
from jax.experimental.pallas import tpu_sc as plsc

NC, NS = 2, 16
NW = NC * NS
CH = 128


def _mesh():
    return plsc.VectorSubcoreMesh(core_axis_name="c", subcore_axis_name="s")


_SC_PARAMS = pltpu.CompilerParams(use_tc_tiling_on_sc=False)


def _wid():
    return lax.axis_index("s") * NC + lax.axis_index("c")


def _gather_rows_kernel(table_hbm, idx_hbm, out_hbm, idx_v, rows_v, sem):
    wid = _wid()
    k = idx_v.shape[0]
    pltpu.sync_copy(idx_hbm.at[wid], idx_v)
    for j in range(k):
        pltpu.async_copy(table_hbm.at[idx_v.at[j]], rows_v, sem).wait()
        pltpu.sync_copy(rows_v, out_hbm.at[pl.ds((wid * k + j) * CH, CH)])


def _scatter_add_kernel(aug_hbm, dst_hbm, zeros_hbm, ones_hbm, out_hbm,
                        idx_v, rows_v, ones_v, acc_msg, acc_deg):
    c = lax.axis_index("c")
    s = lax.axis_index("s")
    wid = s * NC + c
    k = idx_v.shape[0]
    n = out_hbm.shape[1]
    rpt = n // NS
    pltpu.sync_copy(zeros_hbm.at[pl.ds(0, rpt)],
                    acc_msg.at[pl.ds(s * rpt, rpt)])
    pltpu.sync_copy(zeros_hbm.at[pl.ds(0, rpt)],
                    acc_deg.at[pl.ds(s * rpt, rpt)])
    @pl.when(s == 0)
    def _():
        pad = acc_msg.shape[0] - n
        pltpu.sync_copy(zeros_hbm.at[pl.ds(0, pad)],
                        acc_msg.at[pl.ds(n, pad)])
        pltpu.sync_copy(zeros_hbm.at[pl.ds(0, pad)],
                        acc_deg.at[pl.ds(n, pad)])
    pltpu.sync_copy(dst_hbm.at[wid], idx_v)
    pltpu.sync_copy(ones_hbm, ones_v)
    plsc.subcore_barrier()
    for j in range(k):
        pltpu.sync_copy(
            aug_hbm.at[pl.ds((wid * k + j) * CH, CH), pl.ds(0, 16)], rows_v)
        pltpu.sync_copy(rows_v, acc_msg.at[idx_v.at[j]], add=True)
        pltpu.sync_copy(ones_v, acc_deg.at[idx_v.at[j]], add=True)
    plsc.subcore_barrier()
    pltpu.sync_copy(acc_msg.at[pl.ds(s * rpt, rpt)],
                    out_hbm.at[c, pl.ds(s * rpt, rpt), pl.ds(0, 16)])
    pltpu.sync_copy(acc_deg.at[pl.ds(s * rpt, rpt)],
                    out_hbm.at[c, pl.ds(s * rpt, rpt), pl.ds(16, 16)])


def _idx_gather_kernel(src_hbm, dst_hbm, ei_hbm, sidx_hbm, didx_hbm,
                       table_v, ei_v, out_v):
    wid = _wid()
    pt = ei_v.shape[0]
    pltpu.sync_copy(ei_hbm.at[pl.ds(wid * pt, pt)], ei_v)
    for table, out in ((src_hbm, sidx_hbm), (dst_hbm, didx_hbm)):
        pltpu.sync_copy(table, table_v)
        for j in range(pt // 16):
            idx16 = ei_v[pl.ds(j * 16, 16)]
            out_v[pl.ds(j * 16, 16)] = plsc.load_gather(table_v, [idx16])
        pltpu.sync_copy(out_v, out.at[pl.ds(wid * pt, pt)])


def _cls_gather_kernel(h_hbm, aug_hbm, si_hbm, di_hbm, ei_hbm, out_hbm,
                       si_v, di_v, ei_v, rows16_v, rows32_v, sem):
    wid = _wid()
    k = si_v.shape[0]
    pltpu.sync_copy(si_hbm.at[wid], si_v)
    pltpu.sync_copy(di_hbm.at[wid], di_v)
    pltpu.sync_copy(ei_hbm.at[wid], ei_v)
    for j in range(k):
        rows = pl.ds((wid * k + j) * CH, CH)
        pltpu.async_copy(h_hbm.at[si_v.at[j]], rows16_v, sem).wait()
        pltpu.sync_copy(rows16_v, out_hbm.at[rows, pl.ds(0, 16)])
        pltpu.async_copy(h_hbm.at[di_v.at[j]], rows16_v, sem).wait()
        pltpu.sync_copy(rows16_v, out_hbm.at[rows, pl.ds(16, 16)])
        pltpu.async_copy(aug_hbm.at[ei_v.at[j]], rows32_v, sem).wait()
        pltpu.sync_copy(rows32_v.at[pl.ds(0, CH), pl.ds(16, 16)],
                        out_hbm.at[rows, pl.ds(32, 16)])


def _make_mlp_msg_kernel(be, e_valid):
    def body(ef_ref, xs_ref, w1_ref, b1_ref, w2_ref, b2_ref, s2_ref,
             wc1e_ref, out_ref):
        pid = pl.program_id(0)
        gid = jax.lax.broadcasted_iota(jnp.int32, (be, 1), 0) + pid * be
        v = (gid < e_valid).astype(jnp.float32)
        ef = (ef_ref[...] * v).astype(jnp.bfloat16)
        t = jnp.maximum(
            jnp.dot(ef, w1_ref[...], preferred_element_type=jnp.float32)
            + b1_ref[...], 0.0)
        we = (jnp.dot(t.astype(jnp.bfloat16), w2_ref[...],
                      preferred_element_type=jnp.float32)
              + b2_ref[...]).astype(jnp.bfloat16)
        xsb = xs_ref[...].astype(jnp.bfloat16)
        xs_rep = pltpu.repeat(xsb, we.shape[1] // xsb.shape[1], 1)
        msg = jnp.dot(we * xs_rep, s2_ref[...],
                      preferred_element_type=jnp.float32) * v
        ec = jnp.dot(ef, wc1e_ref[...], preferred_element_type=jnp.float32)
        out_ref[...] = jnp.concatenate([msg, ec], axis=1)
    return body


def _mean_relu_kernel(bias_ref, p0_ref, p1_ref, h_ref):
    srow = p0_ref[0] + p1_ref[0]
    agg = srow[:, :16]
    deg = srow[:, 16:17]
    h_ref[...] = jnp.maximum(agg / jnp.maximum(deg, 1.0) + bias_ref[...], 0.0)


def _classifier_kernel(cls_ref, w1s_ref, w1d_ref, bc1_ref, wc2_ref, bc2_ref,
                       out_ref):
    cls = cls_ref[...]
    z = (jnp.dot(cls[:, 0:16], w1s_ref[...],
                 preferred_element_type=jnp.float32)
         + jnp.dot(cls[:, 16:32], w1d_ref[...],
                   preferred_element_type=jnp.float32)
         + cls[:, 32:48] + bc1_ref[...])
    out_ref[...] = jnp.dot(jnp.maximum(z, 0.0), wc2_ref[...],
                           preferred_element_type=jnp.float32) + bc2_ref[...]


def kernel(node_feats, edge_feats, edge_index, edge_indices, W1, b1, W2, b2,
           conv_bias, Wc1, bc1, Wc2, bc2):
    n, in_f = node_feats.shape
    e, ef_f = edge_feats.shape
    h_f = conv_bias.shape[0]
    out_f = Wc2.shape[1]
    nsup = edge_indices.shape[0]

    k_e = -(-e // (NW * CH))
    e_pad = NW * k_e * CH
    k_s = -(-nsup // (NW * CH))
    nsup_pad = NW * k_s * CH
    pt = k_s * CH
    n_acc = n + 16

    src = edge_index[0]
    dst = edge_index[1]
    src_p = jnp.pad(src, (0, e_pad - e))
    dst_p = jnp.pad(dst, (0, e_pad - e), constant_values=n)
    ei_p = jnp.pad(edge_indices, (0, nsup_pad - nsup))

    ih = in_f * h_f
    w2_p = W2.reshape(-1, in_f, h_f).transpose(0, 2, 1).reshape(-1, ih)
    b2_p = b2.reshape(in_f, h_f).T.reshape(ih)
    s2_m = (jnp.arange(ih)[:, None] // in_f == jnp.arange(h_f)[None, :]
            ).astype(jnp.bfloat16)

    xs = pl.kernel(
        _gather_rows_kernel,
        out_type=jax.ShapeDtypeStruct((e_pad, in_f), jnp.float32),
        mesh=_mesh(),
        scratch_types=[
            pltpu.VMEM((k_e, CH), jnp.int32),
            pltpu.VMEM((CH, in_f), jnp.float32),
            pltpu.SemaphoreType.DMA,
        ],
        compiler_params=_SC_PARAMS,
    )(node_feats, src_p.reshape(NW, k_e, CH))

    be = 3072
    aug = pl.pallas_call(
        _make_mlp_msg_kernel(be, e),
        grid=(e_pad // be,),
        in_specs=[
            pl.BlockSpec((be, ef_f), lambda i: (i, 0)),
            pl.BlockSpec((be, in_f), lambda i: (i, 0)),
            pl.BlockSpec(W1.shape, lambda i: (0, 0)),
            pl.BlockSpec((1, b1.shape[0]), lambda i: (0, 0)),
            pl.BlockSpec(w2_p.shape, lambda i: (0, 0)),
            pl.BlockSpec((1, ih), lambda i: (0, 0)),
            pl.BlockSpec(s2_m.shape, lambda i: (0, 0)),
            pl.BlockSpec((ef_f, h_f), lambda i: (0, 0)),
        ],
        out_specs=pl.BlockSpec((be, 32), lambda i: (i, 0)),
        out_shape=jax.ShapeDtypeStruct((e_pad, 32), jnp.float32),
    )(edge_feats, xs, W1.astype(jnp.bfloat16), b1[None, :],
      w2_p.astype(jnp.bfloat16), b2_p[None, :].astype(jnp.bfloat16), s2_m,
      Wc1[2 * h_f:].astype(jnp.bfloat16))

    rpt = n // NS
    partials = pl.kernel(
        _scatter_add_kernel,
        out_type=jax.ShapeDtypeStruct((NC, n, 32), jnp.float32),
        mesh=_mesh(),
        scratch_types=[
            pltpu.VMEM((k_e, CH), jnp.int32),
            pltpu.VMEM((CH, 16), jnp.float32),
            pltpu.VMEM((CH, 16), jnp.float32),
            pltpu.VMEM_SHARED((n_acc, 16), jnp.float32),
            pltpu.VMEM_SHARED((n_acc, 16), jnp.float32),
        ],
        compiler_params=_SC_PARAMS,
    )(aug, dst_p.reshape(NW, k_e, CH), jnp.zeros((rpt, 16), jnp.float32),
      jnp.ones((CH, 16), jnp.float32))

    bn = 2000
    h = pl.pallas_call(
        _mean_relu_kernel,
        grid=(n // bn,),
        in_specs=[
            pl.BlockSpec((1, h_f), lambda i: (0, 0)),
            pl.BlockSpec((1, bn, 32), lambda i: (0, i, 0)),
            pl.BlockSpec((1, bn, 32), lambda i: (1, i, 0)),
        ],
        out_specs=pl.BlockSpec((bn, h_f), lambda i: (i, 0)),
        out_shape=jax.ShapeDtypeStruct((n, h_f), jnp.float32),
    )(conv_bias[None, :], partials, partials)

    s_idx, d_idx = pl.kernel(
        _idx_gather_kernel,
        out_type=(jax.ShapeDtypeStruct((nsup_pad,), jnp.int32),
                  jax.ShapeDtypeStruct((nsup_pad,), jnp.int32)),
        mesh=_mesh(),
        scratch_types=[
            pltpu.VMEM((e_pad,), jnp.int32),
            pltpu.VMEM((pt,), jnp.int32),
            pltpu.VMEM((pt,), jnp.int32),
        ],
        compiler_params=pltpu.CompilerParams(needs_layout_passes=False),
    )(src_p, dst_p, ei_p)

    cls_in = pl.kernel(
        _cls_gather_kernel,
        out_type=jax.ShapeDtypeStruct((nsup_pad, 48), jnp.float32),
        mesh=_mesh(),
        scratch_types=[
            pltpu.VMEM((k_s, CH), jnp.int32),
            pltpu.VMEM((k_s, CH), jnp.int32),
            pltpu.VMEM((k_s, CH), jnp.int32),
            pltpu.VMEM((CH, 16), jnp.float32),
            pltpu.VMEM((CH, 32), jnp.float32),
            pltpu.SemaphoreType.DMA,
        ],
        compiler_params=_SC_PARAMS,
    )(h, aug, s_idx.reshape(NW, k_s, CH), d_idx.reshape(NW, k_s, CH),
      ei_p.reshape(NW, k_s, CH))

    bs = 1024
    logits = pl.pallas_call(
        _classifier_kernel,
        grid=(nsup_pad // bs,),
        in_specs=[
            pl.BlockSpec((bs, 48), lambda i: (i, 0)),
            pl.BlockSpec((h_f, h_f), lambda i: (0, 0)),
            pl.BlockSpec((h_f, h_f), lambda i: (0, 0)),
            pl.BlockSpec((1, h_f), lambda i: (0, 0)),
            pl.BlockSpec(Wc2.shape, lambda i: (0, 0)),
            pl.BlockSpec((1, out_f), lambda i: (0, 0)),
        ],
        out_specs=pl.BlockSpec((bs, out_f), lambda i: (i, 0)),
        out_shape=jax.ShapeDtypeStruct((nsup_pad, out_f), jnp.float32),
    )(cls_in, Wc1[:h_f], Wc1[h_f:2 * h_f], bc1[None, :], Wc2, bc2[None, :])

    return logits[:nsup]


# speedup vs baseline: 2.6445x; 1.0576x over previous
import jax
import jax.numpy as jnp
from jax import lax
from jax.experimental import pallas as pl
from jax.experimental.pallas import tpu as pltpu
from jax.experimental.pallas import tpu_sc as plsc

NC, NS = 2, 16
NW = NC * NS
CH = 128


def _mesh():
    return plsc.VectorSubcoreMesh(core_axis_name="c", subcore_axis_name="s")


_SC_PARAMS = pltpu.CompilerParams(use_tc_tiling_on_sc=False)


def _wid():
    return lax.axis_index("s") * NC + lax.axis_index("c")


def _gather_rows_kernel(table_hbm, idx_hbm, out_hbm, idx_v, rows0, rows1,
                        sem0, sem1):
    wid = _wid()
    k = idx_v.shape[0]
    bufs = (rows0, rows1)
    sems = (sem0, sem1)
    pltpu.sync_copy(idx_hbm.at[wid], idx_v)
    cur = pltpu.async_copy(table_hbm.at[idx_v.at[0]], bufs[0], sems[0])
    for j in range(k):
        nxt = None
        if j + 1 < k:
            nxt = pltpu.async_copy(table_hbm.at[idx_v.at[j + 1]],
                                   bufs[(j + 1) % 2], sems[(j + 1) % 2])
        cur.wait()
        pltpu.sync_copy(bufs[j % 2], out_hbm.at[pl.ds((wid * k + j) * CH, CH)])
        cur = nxt


def _scatter_add_kernel(aug_hbm, dst_hbm, zeros_hbm, ones_hbm, out_hbm,
                        idx_v, rows_v, rows2_v, ones_v, acc_msg, acc_deg,
                        sem0, sem1):
    c = lax.axis_index("c")
    s = lax.axis_index("s")
    wid = s * NC + c
    k = idx_v.shape[0]
    n = out_hbm.shape[1]
    rpt = n // NS
    pltpu.sync_copy(zeros_hbm.at[pl.ds(0, rpt)],
                    acc_msg.at[pl.ds(s * rpt, rpt)])
    pltpu.sync_copy(zeros_hbm.at[pl.ds(0, rpt)],
                    acc_deg.at[pl.ds(s * rpt, rpt)])
    @pl.when(s == 0)
    def _():
        pad = acc_msg.shape[0] - n
        pltpu.sync_copy(zeros_hbm.at[pl.ds(0, pad)],
                        acc_msg.at[pl.ds(n, pad)])
        pltpu.sync_copy(zeros_hbm.at[pl.ds(0, pad)],
                        acc_deg.at[pl.ds(n, pad)])
    pltpu.sync_copy(dst_hbm.at[wid], idx_v)
    pltpu.sync_copy(ones_hbm, ones_v)
    plsc.subcore_barrier()
    bufs = (rows_v, rows2_v)
    sems = (sem0, sem1)
    cur = pltpu.async_copy(
        aug_hbm.at[pl.ds(wid * k * CH, CH), pl.ds(0, 16)], bufs[0], sems[0])
    for j in range(k):
        nxt = None
        if j + 1 < k:
            nxt = pltpu.async_copy(
                aug_hbm.at[pl.ds((wid * k + j + 1) * CH, CH), pl.ds(0, 16)],
                bufs[(j + 1) % 2], sems[(j + 1) % 2])
        cur.wait()
        pltpu.sync_copy(bufs[j % 2], acc_msg.at[idx_v.at[j]], add=True)
        pltpu.sync_copy(ones_v, acc_deg.at[idx_v.at[j]], add=True)
        cur = nxt
    plsc.subcore_barrier()
    pltpu.sync_copy(acc_msg.at[pl.ds(s * rpt, rpt)],
                    out_hbm.at[c, pl.ds(s * rpt, rpt), pl.ds(0, 16)])
    pltpu.sync_copy(acc_deg.at[pl.ds(s * rpt, rpt)],
                    out_hbm.at[c, pl.ds(s * rpt, rpt), pl.ds(16, 16)])


def _idx_gather_kernel(src_hbm, dst_hbm, ei_hbm, sidx_hbm, didx_hbm,
                       table_v, ei_v, out_v):
    wid = _wid()
    pt = ei_v.shape[0]
    pltpu.sync_copy(ei_hbm.at[pl.ds(wid * pt, pt)], ei_v)
    for table, out in ((src_hbm, sidx_hbm), (dst_hbm, didx_hbm)):
        pltpu.sync_copy(table, table_v)
        for j in range(pt // 16):
            idx16 = ei_v[pl.ds(j * 16, 16)]
            out_v[pl.ds(j * 16, 16)] = plsc.load_gather(table_v, [idx16])
        pltpu.sync_copy(out_v, out.at[pl.ds(wid * pt, pt)])


def _cls_gather_kernel(h_hbm, aug_hbm, si_hbm, di_hbm, ei_hbm, out_hbm,
                       si_v, di_v, ei_v, b0, b1, b2, b3, s0, s1, s2, s3):
    wid = _wid()
    k = si_v.shape[0]
    pltpu.sync_copy(si_hbm.at[wid], si_v)
    pltpu.sync_copy(di_hbm.at[wid], di_v)
    pltpu.sync_copy(ei_hbm.at[wid], ei_v)
    rings = {16: ((b0, s0), (b1, s1)), 32: ((b2, s2), (b3, s3))}
    counts = {16: 0, 32: 0}
    work = []
    for j in range(k):
        rows = pl.ds((wid * k + j) * CH, CH)
        work.append((h_hbm, si_v.at[j], 16, pl.ds(0, 16),
                     out_hbm.at[rows, pl.ds(0, 16)]))
        work.append((h_hbm, di_v.at[j], 16, pl.ds(0, 16),
                     out_hbm.at[rows, pl.ds(16, 16)]))
        work.append((aug_hbm, ei_v.at[j], 32, pl.ds(16, 16),
                     out_hbm.at[rows, pl.ds(32, 16)]))
    pending = {}
    for table, idx, wdt, src_cols, dst in work:
        slot = (wdt, counts[wdt] % 2)
        counts[wdt] += 1
        if slot in pending:
            cp, bb, sc, dd = pending.pop(slot)
            cp.wait()
            pltpu.sync_copy(bb.at[pl.ds(0, CH), sc], dd)
        buf, sem = rings[wdt][slot[1]]
        pending[slot] = (pltpu.async_copy(table.at[idx], buf, sem),
                        buf, src_cols, dst)
    for cp, bb, sc, dd in pending.values():
        cp.wait()
        pltpu.sync_copy(bb.at[pl.ds(0, CH), sc], dd)


def _make_mlp_msg_kernel(be, e_valid):
    def body(ef_ref, xs_ref, w1_ref, b1_ref, w2_ref, b2_ref, s2_ref,
             wc1e_ref, out_ref):
        pid = pl.program_id(0)
        gid = jax.lax.broadcasted_iota(jnp.int32, (be, 1), 0) + pid * be
        v = (gid < e_valid).astype(jnp.float32)
        ef = (ef_ref[...] * v).astype(jnp.bfloat16)
        t = jnp.maximum(
            jnp.dot(ef, w1_ref[...], preferred_element_type=jnp.float32)
            + b1_ref[...], 0.0)
        we = (jnp.dot(t.astype(jnp.bfloat16), w2_ref[...],
                      preferred_element_type=jnp.float32)
              + b2_ref[...]).astype(jnp.bfloat16)
        xsb = xs_ref[...].astype(jnp.bfloat16)
        xs_rep = pltpu.repeat(xsb, we.shape[1] // xsb.shape[1], 1)
        msg = jnp.dot(we * xs_rep, s2_ref[...],
                      preferred_element_type=jnp.float32) * v
        ec = jnp.dot(ef, wc1e_ref[...], preferred_element_type=jnp.float32)
        out_ref[...] = jnp.concatenate([msg, ec], axis=1)
    return body


def _mean_relu_kernel(bias_ref, p0_ref, p1_ref, h_ref):
    srow = p0_ref[0] + p1_ref[0]
    agg = srow[:, :16]
    deg = srow[:, 16:17]
    h_ref[...] = jnp.maximum(agg / jnp.maximum(deg, 1.0) + bias_ref[...], 0.0)


def _classifier_kernel(cls_ref, w1s_ref, w1d_ref, bc1_ref, wc2_ref, bc2_ref,
                       out_ref):
    cls = cls_ref[...]
    z = (jnp.dot(cls[:, 0:16], w1s_ref[...],
                 preferred_element_type=jnp.float32)
         + jnp.dot(cls[:, 16:32], w1d_ref[...],
                   preferred_element_type=jnp.float32)
         + cls[:, 32:48] + bc1_ref[...])
    out_ref[...] = jnp.dot(jnp.maximum(z, 0.0), wc2_ref[...],
                           preferred_element_type=jnp.float32) + bc2_ref[...]


def kernel(node_feats, edge_feats, edge_index, edge_indices, W1, b1, W2, b2,
           conv_bias, Wc1, bc1, Wc2, bc2):
    n, in_f = node_feats.shape
    e, ef_f = edge_feats.shape
    h_f = conv_bias.shape[0]
    out_f = Wc2.shape[1]
    nsup = edge_indices.shape[0]

    k_e = -(-e // (NW * CH))
    e_pad = NW * k_e * CH
    k_s = -(-nsup // (NW * CH))
    nsup_pad = NW * k_s * CH
    pt = k_s * CH
    n_acc = n + 16

    src = edge_index[0]
    dst = edge_index[1]
    src_p = jnp.pad(src, (0, e_pad - e))
    dst_p = jnp.pad(dst, (0, e_pad - e), constant_values=n)
    ei_p = jnp.pad(edge_indices, (0, nsup_pad - nsup))

    ih = in_f * h_f
    w2_p = W2.reshape(-1, in_f, h_f).transpose(0, 2, 1).reshape(-1, ih)
    b2_p = b2.reshape(in_f, h_f).T.reshape(ih)
    s2_m = (jnp.arange(ih)[:, None] // in_f == jnp.arange(h_f)[None, :]
            ).astype(jnp.bfloat16)

    xs = pl.kernel(
        _gather_rows_kernel,
        out_type=jax.ShapeDtypeStruct((e_pad, in_f), jnp.float32),
        mesh=_mesh(),
        scratch_types=[
            pltpu.VMEM((k_e, CH), jnp.int32),
            pltpu.VMEM((CH, in_f), jnp.float32),
            pltpu.VMEM((CH, in_f), jnp.float32),
            pltpu.SemaphoreType.DMA,
            pltpu.SemaphoreType.DMA,
        ],
        compiler_params=_SC_PARAMS,
    )(node_feats, src_p.reshape(NW, k_e, CH))

    be = 3072
    aug = pl.pallas_call(
        _make_mlp_msg_kernel(be, e),
        grid=(e_pad // be,),
        in_specs=[
            pl.BlockSpec((be, ef_f), lambda i: (i, 0)),
            pl.BlockSpec((be, in_f), lambda i: (i, 0)),
            pl.BlockSpec(W1.shape, lambda i: (0, 0)),
            pl.BlockSpec((1, b1.shape[0]), lambda i: (0, 0)),
            pl.BlockSpec(w2_p.shape, lambda i: (0, 0)),
            pl.BlockSpec((1, ih), lambda i: (0, 0)),
            pl.BlockSpec(s2_m.shape, lambda i: (0, 0)),
            pl.BlockSpec((ef_f, h_f), lambda i: (0, 0)),
        ],
        out_specs=pl.BlockSpec((be, 32), lambda i: (i, 0)),
        out_shape=jax.ShapeDtypeStruct((e_pad, 32), jnp.float32),
    )(edge_feats, xs, W1.astype(jnp.bfloat16), b1[None, :],
      w2_p.astype(jnp.bfloat16), b2_p[None, :].astype(jnp.bfloat16), s2_m,
      Wc1[2 * h_f:].astype(jnp.bfloat16))

    rpt = n // NS
    partials = pl.kernel(
        _scatter_add_kernel,
        out_type=jax.ShapeDtypeStruct((NC, n, 32), jnp.float32),
        mesh=_mesh(),
        scratch_types=[
            pltpu.VMEM((k_e, CH), jnp.int32),
            pltpu.VMEM((CH, 16), jnp.float32),
            pltpu.VMEM((CH, 16), jnp.float32),
            pltpu.VMEM((CH, 16), jnp.float32),
            pltpu.VMEM_SHARED((n_acc, 16), jnp.float32),
            pltpu.VMEM_SHARED((n_acc, 16), jnp.float32),
            pltpu.SemaphoreType.DMA,
            pltpu.SemaphoreType.DMA,
        ],
        compiler_params=_SC_PARAMS,
    )(aug, dst_p.reshape(NW, k_e, CH), jnp.zeros((rpt, 16), jnp.float32),
      jnp.ones((CH, 16), jnp.float32))

    bn = 2000
    h = pl.pallas_call(
        _mean_relu_kernel,
        grid=(n // bn,),
        in_specs=[
            pl.BlockSpec((1, h_f), lambda i: (0, 0)),
            pl.BlockSpec((1, bn, 32), lambda i: (0, i, 0)),
            pl.BlockSpec((1, bn, 32), lambda i: (1, i, 0)),
        ],
        out_specs=pl.BlockSpec((bn, h_f), lambda i: (i, 0)),
        out_shape=jax.ShapeDtypeStruct((n, h_f), jnp.float32),
    )(conv_bias[None, :], partials, partials)

    s_idx, d_idx = pl.kernel(
        _idx_gather_kernel,
        out_type=(jax.ShapeDtypeStruct((nsup_pad,), jnp.int32),
                  jax.ShapeDtypeStruct((nsup_pad,), jnp.int32)),
        mesh=_mesh(),
        scratch_types=[
            pltpu.VMEM((e_pad,), jnp.int32),
            pltpu.VMEM((pt,), jnp.int32),
            pltpu.VMEM((pt,), jnp.int32),
        ],
        compiler_params=pltpu.CompilerParams(needs_layout_passes=False),
    )(src_p, dst_p, ei_p)

    cls_in = pl.kernel(
        _cls_gather_kernel,
        out_type=jax.ShapeDtypeStruct((nsup_pad, 48), jnp.float32),
        mesh=_mesh(),
        scratch_types=[
            pltpu.VMEM((k_s, CH), jnp.int32),
            pltpu.VMEM((k_s, CH), jnp.int32),
            pltpu.VMEM((k_s, CH), jnp.int32),
            pltpu.VMEM((CH, 16), jnp.float32),
            pltpu.VMEM((CH, 16), jnp.float32),
            pltpu.VMEM((CH, 32), jnp.float32),
            pltpu.VMEM((CH, 32), jnp.float32),
            pltpu.SemaphoreType.DMA,
            pltpu.SemaphoreType.DMA,
            pltpu.SemaphoreType.DMA,
            pltpu.SemaphoreType.DMA,
        ],
        compiler_params=_SC_PARAMS,
    )(h, aug, s_idx.reshape(NW, k_s, CH), d_idx.reshape(NW, k_s, CH),
      ei_p.reshape(NW, k_s, CH))

    bs = 1024
    logits = pl.pallas_call(
        _classifier_kernel,
        grid=(nsup_pad // bs,),
        in_specs=[
            pl.BlockSpec((bs, 48), lambda i: (i, 0)),
            pl.BlockSpec((h_f, h_f), lambda i: (0, 0)),
            pl.BlockSpec((h_f, h_f), lambda i: (0, 0)),
            pl.BlockSpec((1, h_f), lambda i: (0, 0)),
            pl.BlockSpec(Wc2.shape, lambda i: (0, 0)),
            pl.BlockSpec((1, out_f), lambda i: (0, 0)),
        ],
        out_specs=pl.BlockSpec((bs, out_f), lambda i: (i, 0)),
        out_shape=jax.ShapeDtypeStruct((nsup_pad, out_f), jnp.float32),
    )(cls_in, Wc1[:h_f], Wc1[h_f:2 * h_f], bc1[None, :], Wc2, bc2[None, :])

    return logits[:nsup]


# speedup vs baseline: 3.2343x; 1.2230x over previous
import jax
import jax.numpy as jnp
from jax import lax
from jax.experimental import pallas as pl
from jax.experimental.pallas import tpu as pltpu
from jax.experimental.pallas import tpu_sc as plsc

NC, NS = 2, 16
NW = NC * NS
CH = 128


def _mesh():
    return plsc.VectorSubcoreMesh(core_axis_name="c", subcore_axis_name="s")


_SC_PARAMS = pltpu.CompilerParams(use_tc_tiling_on_sc=False)


def _wid():
    return lax.axis_index("s") * NC + lax.axis_index("c")


def _gather_rows_kernel(table_hbm, idx_hbm, out_hbm, idx_v, rows0, rows1,
                        sem0, sem1):
    wid = _wid()
    k = idx_v.shape[0]
    bufs = (rows0, rows1)
    sems = (sem0, sem1)
    q = wid // 8
    sub = wid % 8
    base = sub * k * CH
    width = bufs[0].shape[1]
    pltpu.sync_copy(idx_hbm.at[wid], idx_v)
    cur = pltpu.async_copy(table_hbm.at[idx_v.at[0]], bufs[0], sems[0])
    for j in range(k):
        nxt = None
        if j + 1 < k:
            nxt = pltpu.async_copy(table_hbm.at[idx_v.at[j + 1]],
                                   bufs[(j + 1) % 2], sems[(j + 1) % 2])
        cur.wait()
        pltpu.sync_copy(bufs[j % 2],
                        out_hbm.at[pl.ds(base + j * CH, CH),
                                   pl.ds(q * width, width)])
        cur = nxt


def _scatter_add_kernel(aug_hbm, dst_hbm, zeros_hbm, ones_hbm, out_hbm,
                        idx_v, rows_v, rows2_v, ones_v, acc_msg, acc_deg,
                        sem0, sem1):
    c = lax.axis_index("c")
    s = lax.axis_index("s")
    wid = s * NC + c
    k = idx_v.shape[0]
    n = out_hbm.shape[1]
    rpt = n // NS
    pltpu.sync_copy(zeros_hbm.at[pl.ds(0, rpt)],
                    acc_msg.at[pl.ds(s * rpt, rpt)])
    pltpu.sync_copy(zeros_hbm.at[pl.ds(0, rpt)],
                    acc_deg.at[pl.ds(s * rpt, rpt)])
    @pl.when(s == 0)
    def _():
        pad = acc_msg.shape[0] - n
        pltpu.sync_copy(zeros_hbm.at[pl.ds(0, pad)],
                        acc_msg.at[pl.ds(n, pad)])
        pltpu.sync_copy(zeros_hbm.at[pl.ds(0, pad)],
                        acc_deg.at[pl.ds(n, pad)])
    pltpu.sync_copy(dst_hbm.at[wid], idx_v)
    pltpu.sync_copy(ones_hbm, ones_v)
    plsc.subcore_barrier()
    bufs = (rows_v, rows2_v)
    sems = (sem0, sem1)
    q = wid // 8
    base = (wid % 8) * k * CH
    cur = pltpu.async_copy(
        aug_hbm.at[pl.ds(base, CH), pl.ds(q * 32, 16)], bufs[0], sems[0])
    for j in range(k):
        nxt = None
        if j + 1 < k:
            nxt = pltpu.async_copy(
                aug_hbm.at[pl.ds(base + (j + 1) * CH, CH),
                           pl.ds(q * 32, 16)],
                bufs[(j + 1) % 2], sems[(j + 1) % 2])
        cur.wait()
        pltpu.sync_copy(bufs[j % 2], acc_msg.at[idx_v.at[j]], add=True)
        pltpu.sync_copy(ones_v, acc_deg.at[idx_v.at[j]], add=True)
        cur = nxt
    plsc.subcore_barrier()
    pltpu.sync_copy(acc_msg.at[pl.ds(s * rpt, rpt)],
                    out_hbm.at[c, pl.ds(s * rpt, rpt), pl.ds(0, 16)])
    pltpu.sync_copy(acc_deg.at[pl.ds(s * rpt, rpt)],
                    out_hbm.at[c, pl.ds(s * rpt, rpt), pl.ds(16, 16)])


def _idx_gather_kernel(src_hbm, dst_hbm, ei_hbm, sidx_hbm, didx_hbm,
                       table_v, ei_v, out_v):
    wid = _wid()
    pt = ei_v.shape[0]
    pltpu.sync_copy(ei_hbm.at[pl.ds(wid * pt, pt)], ei_v)
    for table, out in ((src_hbm, sidx_hbm), (dst_hbm, didx_hbm)):
        pltpu.sync_copy(table, table_v)
        for j in range(pt // 16):
            idx16 = ei_v[pl.ds(j * 16, 16)]
            out_v[pl.ds(j * 16, 16)] = plsc.load_gather(table_v, [idx16])
        pltpu.sync_copy(out_v, out.at[pl.ds(wid * pt, pt)])


def _cls_gather_kernel(h_hbm, ecf_hbm, si_hbm, di_hbm, er_hbm, out_hbm,
                       si_v, di_v, er_v, b0, b1, b2, b3, s0, s1, s2, s3):
    wid = _wid()
    k = si_v.shape[0]
    pltpu.sync_copy(si_hbm.at[wid], si_v)
    pltpu.sync_copy(di_hbm.at[wid], di_v)
    pltpu.sync_copy(er_hbm.at[wid], er_v)
    bufs = (b0, b1, b2, b3)
    sems = (s0, s1, s2, s3)
    work = []
    for j in range(k):
        rows = pl.ds((wid * k + j) * CH, CH)
        work.append((h_hbm, si_v.at[j], out_hbm.at[rows, pl.ds(0, 16)]))
        work.append((h_hbm, di_v.at[j], out_hbm.at[rows, pl.ds(16, 16)]))
        work.append((ecf_hbm, er_v.at[j], out_hbm.at[rows, pl.ds(32, 16)]))
    pending = []
    for i, (table, idx, dst) in enumerate(work):
        if len(pending) == 4:
            cp, bb, dd = pending.pop(0)
            cp.wait()
            pltpu.sync_copy(bb, dd)
        b = i % 4
        pending.append((pltpu.async_copy(table.at[idx], bufs[b], sems[b]),
                        bufs[b], dst))
    for cp, bb, dd in pending:
        cp.wait()
        pltpu.sync_copy(bb, dd)


def _make_mlp_msg_kernel(be, q_rows, e_valid):
    def body(ef0_ref, ef1_ref, ef2_ref, ef3_ref, xs_ref, w1_ref, b1_ref,
             w2_ref, b2_ref, s2_ref, wc1e_ref, out_ref):
        pid = pl.program_id(0)
        xs_all = xs_ref[...]
        parts = []
        for q, efq in enumerate((ef0_ref, ef1_ref, ef2_ref, ef3_ref)):
            gid = (jax.lax.broadcasted_iota(jnp.int32, (be, 1), 0)
                   + q * q_rows + pid * be)
            v = (gid < e_valid).astype(jnp.float32)
            ef = (efq[...] * v).astype(jnp.bfloat16)
            t = jnp.maximum(
                jnp.dot(ef, w1_ref[...], preferred_element_type=jnp.float32)
                + b1_ref[...], 0.0)
            we = (jnp.dot(t.astype(jnp.bfloat16), w2_ref[...],
                          preferred_element_type=jnp.float32)
                  + b2_ref[...]).astype(jnp.bfloat16)
            xsb = xs_all[:, q * 32:(q + 1) * 32].astype(jnp.bfloat16)
            xs_rep = pltpu.repeat(xsb, we.shape[1] // xsb.shape[1], 1)
            msg = jnp.dot(we * xs_rep, s2_ref[...],
                          preferred_element_type=jnp.float32) * v
            ec = jnp.dot(ef, wc1e_ref[...],
                         preferred_element_type=jnp.float32)
            parts.append(msg)
            parts.append(ec)
        out_ref[...] = jnp.concatenate(parts, axis=1)
    return body


def _mean_relu_kernel(bias_ref, p0_ref, p1_ref, h_ref):
    srow = p0_ref[0] + p1_ref[0]
    agg = srow[:, :16]
    deg = srow[:, 16:17]
    h_ref[...] = jnp.maximum(agg / jnp.maximum(deg, 1.0) + bias_ref[...], 0.0)


def _classifier_kernel(cls_ref, w1s_ref, w1d_ref, bc1_ref, wc2_ref, bc2_ref,
                       out_ref):
    cls = cls_ref[...]
    z = (jnp.dot(cls[:, 0:16], w1s_ref[...],
                 preferred_element_type=jnp.float32)
         + jnp.dot(cls[:, 16:32], w1d_ref[...],
                   preferred_element_type=jnp.float32)
         + cls[:, 32:48] + bc1_ref[...])
    out_ref[...] = jnp.dot(jnp.maximum(z, 0.0), wc2_ref[...],
                           preferred_element_type=jnp.float32) + bc2_ref[...]


def kernel(node_feats, edge_feats, edge_index, edge_indices, W1, b1, W2, b2,
           conv_bias, Wc1, bc1, Wc2, bc2):
    n, in_f = node_feats.shape
    e, ef_f = edge_feats.shape
    h_f = conv_bias.shape[0]
    out_f = Wc2.shape[1]
    nsup = edge_indices.shape[0]

    k_e = -(-e // (NW * CH))
    e_pad = NW * k_e * CH
    k_s = -(-nsup // (NW * CH))
    nsup_pad = NW * k_s * CH
    pt = k_s * CH
    n_acc = n + 16

    src = edge_index[0]
    dst = edge_index[1]
    src_p = jnp.pad(src, (0, e_pad - e))
    dst_p = jnp.pad(dst, (0, e_pad - e), constant_values=n)
    ei_p = jnp.pad(edge_indices, (0, nsup_pad - nsup))

    ih = in_f * h_f
    w2_p = W2.reshape(-1, in_f, h_f).transpose(0, 2, 1).reshape(-1, ih)
    b2_p = b2.reshape(in_f, h_f).T.reshape(ih)
    s2_m = (jnp.arange(ih)[:, None] // in_f == jnp.arange(h_f)[None, :]
            ).astype(jnp.bfloat16)

    q_rows = e_pad // 4
    xs = pl.kernel(
        _gather_rows_kernel,
        out_type=jax.ShapeDtypeStruct((q_rows, 4 * in_f), jnp.float32),
        mesh=_mesh(),
        scratch_types=[
            pltpu.VMEM((k_e, CH), jnp.int32),
            pltpu.VMEM((CH, in_f), jnp.float32),
            pltpu.VMEM((CH, in_f), jnp.float32),
            pltpu.SemaphoreType.DMA,
            pltpu.SemaphoreType.DMA,
        ],
        compiler_params=_SC_PARAMS,
    )(node_feats, src_p.reshape(NW, k_e, CH))

    be = 3072
    ef_specs = [
        pl.BlockSpec((be, ef_f), lambda i, q=q: (q * (q_rows // be) + i, 0))
        for q in range(4)
    ]
    aug = pl.pallas_call(
        _make_mlp_msg_kernel(be, q_rows, e),
        grid=(q_rows // be,),
        in_specs=ef_specs + [
            pl.BlockSpec((be, 4 * in_f), lambda i: (i, 0)),
            pl.BlockSpec(W1.shape, lambda i: (0, 0)),
            pl.BlockSpec((1, b1.shape[0]), lambda i: (0, 0)),
            pl.BlockSpec(w2_p.shape, lambda i: (0, 0)),
            pl.BlockSpec((1, ih), lambda i: (0, 0)),
            pl.BlockSpec(s2_m.shape, lambda i: (0, 0)),
            pl.BlockSpec((ef_f, h_f), lambda i: (0, 0)),
        ],
        out_specs=pl.BlockSpec((be, 128), lambda i: (i, 0)),
        out_shape=jax.ShapeDtypeStruct((q_rows, 128), jnp.float32),
    )(edge_feats, edge_feats, edge_feats, edge_feats, xs,
      W1.astype(jnp.bfloat16), b1[None, :], w2_p.astype(jnp.bfloat16),
      b2_p[None, :].astype(jnp.bfloat16), s2_m,
      Wc1[2 * h_f:].astype(jnp.bfloat16))

    rpt = n // NS
    partials = pl.kernel(
        _scatter_add_kernel,
        out_type=jax.ShapeDtypeStruct((NC, n, 32), jnp.float32),
        mesh=_mesh(),
        scratch_types=[
            pltpu.VMEM((k_e, CH), jnp.int32),
            pltpu.VMEM((CH, 16), jnp.float32),
            pltpu.VMEM((CH, 16), jnp.float32),
            pltpu.VMEM((CH, 16), jnp.float32),
            pltpu.VMEM_SHARED((n_acc, 16), jnp.float32),
            pltpu.VMEM_SHARED((n_acc, 16), jnp.float32),
            pltpu.SemaphoreType.DMA,
            pltpu.SemaphoreType.DMA,
        ],
        compiler_params=_SC_PARAMS,
    )(aug, dst_p.reshape(NW, k_e, CH), jnp.zeros((rpt, 16), jnp.float32),
      jnp.ones((CH, 16), jnp.float32))

    bn = 2000
    h = pl.pallas_call(
        _mean_relu_kernel,
        grid=(n // bn,),
        in_specs=[
            pl.BlockSpec((1, h_f), lambda i: (0, 0)),
            pl.BlockSpec((1, bn, 32), lambda i: (0, i, 0)),
            pl.BlockSpec((1, bn, 32), lambda i: (1, i, 0)),
        ],
        out_specs=pl.BlockSpec((bn, h_f), lambda i: (i, 0)),
        out_shape=jax.ShapeDtypeStruct((n, h_f), jnp.float32),
    )(conv_bias[None, :], partials, partials)

    s_idx, d_idx = pl.kernel(
        _idx_gather_kernel,
        out_type=(jax.ShapeDtypeStruct((nsup_pad,), jnp.int32),
                  jax.ShapeDtypeStruct((nsup_pad,), jnp.int32)),
        mesh=_mesh(),
        scratch_types=[
            pltpu.VMEM((e_pad,), jnp.int32),
            pltpu.VMEM((pt,), jnp.int32),
            pltpu.VMEM((pt,), jnp.int32),
        ],
        compiler_params=pltpu.CompilerParams(needs_layout_passes=False),
    )(src_p, dst_p, ei_p)

    ec_rows = (ei_p % q_rows) * 8 + (ei_p // q_rows) * 2 + 1
    cls_in = pl.kernel(
        _cls_gather_kernel,
        out_type=jax.ShapeDtypeStruct((nsup_pad, 48), jnp.float32),
        mesh=_mesh(),
        scratch_types=[
            pltpu.VMEM((k_s, CH), jnp.int32),
            pltpu.VMEM((k_s, CH), jnp.int32),
            pltpu.VMEM((k_s, CH), jnp.int32),
            pltpu.VMEM((CH, 16), jnp.float32),
            pltpu.VMEM((CH, 16), jnp.float32),
            pltpu.VMEM((CH, 16), jnp.float32),
            pltpu.VMEM((CH, 16), jnp.float32),
            pltpu.SemaphoreType.DMA,
            pltpu.SemaphoreType.DMA,
            pltpu.SemaphoreType.DMA,
            pltpu.SemaphoreType.DMA,
        ],
        compiler_params=_SC_PARAMS,
    )(h, aug.reshape(q_rows * 8, 16), s_idx.reshape(NW, k_s, CH),
      d_idx.reshape(NW, k_s, CH), ec_rows.reshape(NW, k_s, CH))

    bs = 1024
    logits = pl.pallas_call(
        _classifier_kernel,
        grid=(nsup_pad // bs,),
        in_specs=[
            pl.BlockSpec((bs, 48), lambda i: (i, 0)),
            pl.BlockSpec((h_f, h_f), lambda i: (0, 0)),
            pl.BlockSpec((h_f, h_f), lambda i: (0, 0)),
            pl.BlockSpec((1, h_f), lambda i: (0, 0)),
            pl.BlockSpec(Wc2.shape, lambda i: (0, 0)),
            pl.BlockSpec((1, out_f), lambda i: (0, 0)),
        ],
        out_specs=pl.BlockSpec((bs, out_f), lambda i: (i, 0)),
        out_shape=jax.ShapeDtypeStruct((nsup_pad, out_f), jnp.float32),
    )(cls_in, Wc1[:h_f], Wc1[h_f:2 * h_f], bc1[None, :], Wc2, bc2[None, :])

    return logits[:nsup]
